# Initial kernel scaffold; baseline (speedup 1.0000x reference)
import jax
import jax.numpy as jnp
from jax.experimental import pallas as pl


def kernel(logits):
    raise NotImplementedError("write your pallas kernel here")


# speedup vs baseline: 23.4572x; 23.4572x over previous
import functools

import jax
import jax.numpy as jnp
from jax import lax
from jax.experimental import pallas as pl

_TOPK = 64


def _mono_u32(x):
    u = lax.bitcast_convert_type(x, jnp.uint32)
    neg = (u >> 31).astype(jnp.bool_)
    return jnp.where(neg, ~u, u | jnp.uint32(0x80000000))


def _inv_mono_u32(m):
    pos = (m >> 31).astype(jnp.bool_)
    bits = jnp.where(pos, m & jnp.uint32(0x7FFFFFFF), ~m)
    return lax.bitcast_convert_type(bits, jnp.float32)


def _topk_mask_kernel(x_ref, o_ref, *, k):
    x = x_ref[0]
    u = _mono_u32(x)

    def body(i, prefix):
        b = (jnp.uint32(31) - i.astype(jnp.uint32))
        cand = prefix | (jnp.uint32(1) << b)
        cnt = jnp.sum((u >= cand).astype(jnp.float32), axis=-1, keepdims=True)
        return jnp.where(cnt >= k, cand, prefix)

    prefix = lax.fori_loop(
        0, 32, body, jnp.zeros((x.shape[0], 1), jnp.uint32))

    thresh = _inv_mono_u32(prefix)
    ge = u >= prefix
    gef = ge.astype(jnp.float32)
    cnt = jnp.sum(gef, axis=-1, keepdims=True)
    ssum = jnp.sum(x * gef, axis=-1, keepdims=True)
    ssum = ssum - (cnt - k) * thresh
    o_ref[0] = (x * gef) / (ssum + 1e-12)


def kernel(logits):
    C, S, m = logits.shape
    k = min(_TOPK, m)
    return pl.pallas_call(
        functools.partial(_topk_mask_kernel, k=k),
        grid=(C,),
        in_specs=[pl.BlockSpec((1, S, m), lambda c: (c, 0, 0))],
        out_specs=pl.BlockSpec((1, S, m), lambda c: (c, 0, 0)),
        out_shape=jax.ShapeDtypeStruct((C, S, m), jnp.float32),
    )(logits)
